# Initial kernel scaffold; baseline (speedup 1.0000x reference)
import jax
import jax.numpy as jnp
from jax.experimental import pallas as pl


def kernel(node_features, node_masks, W1, b1, W2, b2, W3, b3):
    raise NotImplementedError("write your pallas kernel here")


# speedup vs baseline: 1.1439x; 1.1439x over previous
import functools

import jax
import jax.numpy as jnp
from jax.experimental import pallas as pl


_TI = 64


def _edge_kernel(x_ref, m_ref, w1a_ref, w1b_ref, b1_ref, w2_ref, b2_ref,
                 w3_ref, b3_ref, out_ref):
    n = x_ref.shape[1]
    x = x_ref[0]
    w1a = w1a_ref[...]
    w1b = w1b_ref[...]
    a = jnp.dot(x, w1a, preferred_element_type=jnp.float32) + b1_ref[...]
    b = jnp.dot(x, w1b, preferred_element_type=jnp.float32)

    w2 = w2_ref[...]
    b2 = b2_ref[...]
    w3 = w3_ref[...]
    b3 = b3_ref[0, 0]

    rows = []
    for i0 in range(0, n, _TI):
        h1 = jnp.maximum(a[i0:i0 + _TI, None, :] + b[None, :, :], 0.0)
        h1 = h1.reshape(_TI * n, 64)
        h2 = jnp.maximum(
            jnp.dot(h1, w2, preferred_element_type=jnp.float32) + b2, 0.0)
        z = jnp.dot(h2, w3, preferred_element_type=jnp.float32)
        rows.append(z.reshape(_TI, n))
    score = jax.nn.sigmoid(jnp.concatenate(rows, axis=0) + b3)

    m = m_ref[0, 0]
    pair = m[:, None] * m[None, :]
    ri = jax.lax.broadcasted_iota(jnp.int32, (n, n), 0)
    ci = jax.lax.broadcasted_iota(jnp.int32, (n, n), 1)
    adj = jnp.where(ri == ci, 0.0, score * pair)
    out_ref[0] = (adj + adj.T) * 0.5


@jax.jit
def kernel(node_features, node_masks, W1, b1, W2, b2, W3, b3):
    B, N, F = node_features.shape
    w1a = W1[:, :F].T
    w1b = W1[:, F:].T
    b1r = b1.reshape(1, 64)
    w2 = W2.T
    b2r = b2.reshape(1, 32)
    w3 = W3.T
    b3r = b3.reshape(1, 1)
    masks = node_masks.astype(jnp.float32).reshape(B, 1, N)

    full = lambda shape: pl.BlockSpec(shape, lambda i: (0,) * len(shape))
    out = pl.pallas_call(
        _edge_kernel,
        grid=(B,),
        in_specs=[
            pl.BlockSpec((1, N, F), lambda i: (i, 0, 0)),
            pl.BlockSpec((1, 1, N), lambda i: (i, 0, 0)),
            full((F, 64)),
            full((F, 64)),
            full((1, 64)),
            full((64, 32)),
            full((1, 32)),
            full((32, 1)),
            full((1, 1)),
        ],
        out_specs=pl.BlockSpec((1, N, N), lambda i: (i, 0, 0)),
        out_shape=jax.ShapeDtypeStruct((B, N, N), jnp.float32),
    )(node_features, masks, w1a, w1b, b1r, w2, b2r, w3, b3r)
    return out


# speedup vs baseline: 1.4734x; 1.2881x over previous
import jax
import jax.numpy as jnp
from jax.experimental import pallas as pl


_TI = 128


def _edge_kernel(xt_ref, m_ref, w1a_ref, w1b_ref, b1_ref, w2_ref, b2_ref,
                 w3_ref, b3_ref, out_ref):
    n = xt_ref.shape[2]
    xt = xt_ref[0]
    at = jnp.dot(w1a_ref[...], xt,
                 preferred_element_type=jnp.float32) + b1_ref[...]
    bt = jnp.dot(w1b_ref[...], xt,
                 preferred_element_type=jnp.float32)

    w2 = w2_ref[...]
    b2 = b2_ref[...]
    w3 = w3_ref[...]
    b3 = b3_ref[0, 0]

    rows = []
    for i0 in range(0, n, _TI):
        ab = at[:, i0:i0 + _TI]
        h1 = jnp.maximum(ab[:, :, None] + bt[:, None, :], 0.0)
        h1 = h1.reshape(64, _TI * n)
        h2 = jnp.maximum(
            jnp.dot(w2, h1, preferred_element_type=jnp.float32) + b2, 0.0)
        z = jnp.dot(w3, h2, preferred_element_type=jnp.float32)
        rows.append(z.reshape(_TI, n))
    score = jax.nn.sigmoid(jnp.concatenate(rows, axis=0) + b3)

    m = m_ref[0, 0]
    pair = m[:, None] * m[None, :]
    ri = jax.lax.broadcasted_iota(jnp.int32, (n, n), 0)
    ci = jax.lax.broadcasted_iota(jnp.int32, (n, n), 1)
    adj = jnp.where(ri == ci, 0.0, score * pair)
    out_ref[0] = (adj + adj.T) * 0.5


@jax.jit
def kernel(node_features, node_masks, W1, b1, W2, b2, W3, b3):
    B, N, F = node_features.shape
    xt = jnp.swapaxes(node_features, 1, 2)
    w1a = W1[:, :F]
    w1b = W1[:, F:]
    b1r = b1.reshape(64, 1)
    b2r = b2.reshape(32, 1)
    b3r = b3.reshape(1, 1)
    masks = node_masks.astype(jnp.float32).reshape(B, 1, N)

    full = lambda shape: pl.BlockSpec(shape, lambda i: (0,) * len(shape))
    out = pl.pallas_call(
        _edge_kernel,
        grid=(B,),
        in_specs=[
            pl.BlockSpec((1, F, N), lambda i: (i, 0, 0)),
            pl.BlockSpec((1, 1, N), lambda i: (i, 0, 0)),
            full((64, F)),
            full((64, F)),
            full((64, 1)),
            full((32, 64)),
            full((32, 1)),
            full((1, 32)),
            full((1, 1)),
        ],
        out_specs=pl.BlockSpec((1, N, N), lambda i: (i, 0, 0)),
        out_shape=jax.ShapeDtypeStruct((B, N, N), jnp.float32),
    )(xt, masks, w1a, w1b, b1r, W2, b2r, W3, b3r)
    return out


# speedup vs baseline: 2.6196x; 1.7780x over previous
import jax
import jax.numpy as jnp
from jax.experimental import pallas as pl


_TI = 128


def _edge_kernel(xt_ref, m_ref, w1a_ref, w1b_ref, b1_ref, w2_ref, b2_ref,
                 w3_ref, b3_ref, out_ref):
    n = xt_ref.shape[2]
    xt = xt_ref[0]
    at = jnp.dot(w1a_ref[...], xt,
                 preferred_element_type=jnp.float32) + b1_ref[...]
    bt = jnp.dot(w1b_ref[...], xt,
                 preferred_element_type=jnp.float32)

    w2 = w2_ref[...]
    b2 = b2_ref[...]
    w3 = w3_ref[...]
    b3 = b3_ref[0, 0]

    rows = []
    for i0 in range(0, n, _TI):
        chunks = [jnp.maximum(at[:, ii:ii + 1] + bt, 0.0)
                  for ii in range(i0, i0 + _TI)]
        h1 = jnp.concatenate(chunks, axis=1)
        h2 = jnp.maximum(
            jnp.dot(w2, h1, preferred_element_type=jnp.float32) + b2, 0.0)
        z = jnp.dot(w3, h2, preferred_element_type=jnp.float32)
        rows.append(z.reshape(_TI, n))
    score = jax.nn.sigmoid(jnp.concatenate(rows, axis=0) + b3)

    m = m_ref[0, 0]
    pair = m[:, None] * m[None, :]
    ri = jax.lax.broadcasted_iota(jnp.int32, (n, n), 0)
    ci = jax.lax.broadcasted_iota(jnp.int32, (n, n), 1)
    adj = jnp.where(ri == ci, 0.0, score * pair)
    out_ref[0] = (adj + adj.T) * 0.5


@jax.jit
def kernel(node_features, node_masks, W1, b1, W2, b2, W3, b3):
    B, N, F = node_features.shape
    xt = jnp.swapaxes(node_features, 1, 2)
    w1a = W1[:, :F]
    w1b = W1[:, F:]
    b1r = b1.reshape(64, 1)
    b2r = b2.reshape(32, 1)
    b3r = b3.reshape(1, 1)
    masks = node_masks.astype(jnp.float32).reshape(B, 1, N)

    full = lambda shape: pl.BlockSpec(shape, lambda i: (0,) * len(shape))
    out = pl.pallas_call(
        _edge_kernel,
        grid=(B,),
        in_specs=[
            pl.BlockSpec((1, F, N), lambda i: (i, 0, 0)),
            pl.BlockSpec((1, 1, N), lambda i: (i, 0, 0)),
            full((64, F)),
            full((64, F)),
            full((64, 1)),
            full((32, 64)),
            full((32, 1)),
            full((1, 32)),
            full((1, 1)),
        ],
        out_specs=pl.BlockSpec((1, N, N), lambda i: (i, 0, 0)),
        out_shape=jax.ShapeDtypeStruct((B, N, N), jnp.float32),
    )(xt, masks, w1a, w1b, b1r, W2, b2r, W3, b3r)
    return out


# speedup vs baseline: 3.2559x; 1.2429x over previous
import jax
import jax.numpy as jnp
from jax.experimental import pallas as pl


_TI = 128


def _edge_kernel(xt_ref, m_ref, w1a_ref, w1b_ref, b1_ref, w2_ref, b2_ref,
                 w3_ref, b3_ref, out_ref):
    n = xt_ref.shape[2]
    xt = xt_ref[0]
    at = jnp.dot(w1a_ref[...], xt,
                 preferred_element_type=jnp.float32) + b1_ref[...]
    bt = jnp.dot(w1b_ref[...], xt,
                 preferred_element_type=jnp.float32)

    at16 = at.astype(jnp.bfloat16)
    bt16 = bt.astype(jnp.bfloat16)

    w2 = w2_ref[...]
    b2 = b2_ref[...]
    w3 = w3_ref[...]
    b3 = b3_ref[0, 0]

    rows = []
    for i0 in range(0, n, _TI):
        chunks = [jnp.maximum(at16[:, ii:ii + 1] + bt16, 0)
                  for ii in range(i0, i0 + _TI)]
        h1 = jnp.concatenate(chunks, axis=1)
        h2 = jnp.maximum(
            jnp.dot(w2, h1, preferred_element_type=jnp.float32) + b2, 0.0)
        z = jnp.dot(w3, h2, preferred_element_type=jnp.float32)
        rows.append(z.reshape(_TI, n))
    score = jax.nn.sigmoid(jnp.concatenate(rows, axis=0) + b3)

    m = m_ref[0, 0]
    pair = m[:, None] * m[None, :]
    ri = jax.lax.broadcasted_iota(jnp.int32, (n, n), 0)
    ci = jax.lax.broadcasted_iota(jnp.int32, (n, n), 1)
    adj = jnp.where(ri == ci, 0.0, score * pair)
    out_ref[0] = (adj + adj.T) * 0.5


@jax.jit
def kernel(node_features, node_masks, W1, b1, W2, b2, W3, b3):
    B, N, F = node_features.shape
    xt = jnp.swapaxes(node_features, 1, 2)
    w1a = W1[:, :F]
    w1b = W1[:, F:]
    w2_16 = W2.astype(jnp.bfloat16)
    b1r = b1.reshape(64, 1)
    b2r = b2.reshape(32, 1)
    b3r = b3.reshape(1, 1)
    masks = node_masks.astype(jnp.float32).reshape(B, 1, N)

    full = lambda shape: pl.BlockSpec(shape, lambda i: (0,) * len(shape))
    out = pl.pallas_call(
        _edge_kernel,
        grid=(B,),
        in_specs=[
            pl.BlockSpec((1, F, N), lambda i: (i, 0, 0)),
            pl.BlockSpec((1, 1, N), lambda i: (i, 0, 0)),
            full((64, F)),
            full((64, F)),
            full((64, 1)),
            full((32, 64)),
            full((32, 1)),
            full((1, 32)),
            full((1, 1)),
        ],
        out_specs=pl.BlockSpec((1, N, N), lambda i: (i, 0, 0)),
        out_shape=jax.ShapeDtypeStruct((B, N, N), jnp.float32),
    )(xt, masks, w1a, w1b, b1r, w2_16, b2r, W3, b3r)
    return out


# speedup vs baseline: 3.3247x; 1.0211x over previous
import jax
import jax.numpy as jnp
from jax.experimental import pallas as pl


_TI = 128


def _edge_kernel(xt_ref, m_ref, w1a_ref, w1b_ref, b1_ref, w2_ref, b2_ref,
                 w3_ref, b3_ref, out_ref):
    n = xt_ref.shape[2]
    xt = xt_ref[0]
    at = jnp.dot(w1a_ref[...], xt,
                 preferred_element_type=jnp.float32) + b1_ref[...]
    bt = jnp.dot(w1b_ref[...], xt,
                 preferred_element_type=jnp.float32)

    at16 = at.astype(jnp.bfloat16)
    bt16 = bt.astype(jnp.bfloat16)

    w2 = w2_ref[...]
    b2 = b2_ref[...]
    w3 = w3_ref[...]
    b3 = b3_ref[0, 0]

    rows = []
    for i0 in range(0, n, _TI):
        chunks = [jnp.maximum(at16[:, ii:ii + 1] + bt16, 0)
                  for ii in range(i0, i0 + _TI)]
        h1 = jnp.concatenate(chunks, axis=1)
        h2 = jnp.maximum(
            jnp.dot(w2, h1, preferred_element_type=jnp.float32) + b2, 0.0)
        z = jnp.dot(w3, h2, preferred_element_type=jnp.float32)
        rows.append(z.astype(jnp.bfloat16).reshape(_TI, n))
    zmat = jnp.concatenate(rows, axis=0).astype(jnp.float32)
    score = jax.nn.sigmoid(zmat + b3)

    m = m_ref[0, 0]
    pair = m[:, None] * m[None, :]
    ri = jax.lax.broadcasted_iota(jnp.int32, (n, n), 0)
    ci = jax.lax.broadcasted_iota(jnp.int32, (n, n), 1)
    adj = jnp.where(ri == ci, 0.0, score * pair)
    out_ref[0] = (adj + adj.T) * 0.5


@jax.jit
def kernel(node_features, node_masks, W1, b1, W2, b2, W3, b3):
    B, N, F = node_features.shape
    xt = jnp.swapaxes(node_features, 1, 2)
    w1a = W1[:, :F]
    w1b = W1[:, F:]
    w2_16 = W2.astype(jnp.bfloat16)
    b1r = b1.reshape(64, 1)
    b2r = b2.reshape(32, 1)
    b3r = b3.reshape(1, 1)
    masks = node_masks.astype(jnp.float32).reshape(B, 1, N)

    full = lambda shape: pl.BlockSpec(shape, lambda i: (0,) * len(shape))
    out = pl.pallas_call(
        _edge_kernel,
        grid=(B,),
        in_specs=[
            pl.BlockSpec((1, F, N), lambda i: (i, 0, 0)),
            pl.BlockSpec((1, 1, N), lambda i: (i, 0, 0)),
            full((64, F)),
            full((64, F)),
            full((64, 1)),
            full((32, 64)),
            full((32, 1)),
            full((1, 32)),
            full((1, 1)),
        ],
        out_specs=pl.BlockSpec((1, N, N), lambda i: (i, 0, 0)),
        out_shape=jax.ShapeDtypeStruct((B, N, N), jnp.float32),
    )(xt, masks, w1a, w1b, b1r, w2_16, b2r, W3, b3r)
    return out
